# Initial kernel scaffold; baseline (speedup 1.0000x reference)
import jax
import jax.numpy as jnp
from jax.experimental import pallas as pl


def kernel(imm, regs, mem_reg0, mem_reg1, mem_imm0, mem_imm1, mnemic, mnemic_idx, operand_idx, emb, W_imm1, b_imm1, W_imm2, b_imm2, W_reg1, b_reg1, W_reg2, b_reg2, W_mem1, b_mem1, W_mem2, b_mem2, W_ins1, b_ins1, W_ins2, b_ins2):
    raise NotImplementedError("write your pallas kernel here")


# speedup vs baseline: 6.8731x; 6.8731x over previous
import functools

import jax
import jax.numpy as jnp
from jax import lax
from jax.experimental import pallas as pl
from jax.experimental.pallas import tpu as pltpu
from jax.experimental.pallas import tpu_sc as plsc

_NC = 2
_NS = 16
_NW = _NC * _NS
_CH = 128

_H = 128
_N_REG = 100000
_N_IMM = 50000
_N_MEM = 50000
_N_OPS = _N_REG + _N_IMM + _N_MEM


def _wid():
    return lax.axis_index("s") * _NC + lax.axis_index("c")


def _sc_gather_body(n_pad, table, idx, out, idx_v, rows_v, sem):
    b_per_w = n_pad // _NW
    n_chunks = b_per_w // _CH
    base = _wid() * b_per_w

    def step(j, carry):
        off = base + j * _CH
        pltpu.sync_copy(idx.at[pl.ds(off, _CH)], idx_v)
        pltpu.async_copy(table.at[idx_v], rows_v, sem).wait()
        pltpu.sync_copy(rows_v, out.at[pl.ds(off, _CH)])
        return carry

    lax.fori_loop(0, n_chunks, step, 0, unroll=False)


def _sc_gather(table, idx):
    n_pad = idx.shape[0]
    mesh = plsc.VectorSubcoreMesh(core_axis_name="c", subcore_axis_name="s")
    return pl.kernel(
        functools.partial(_sc_gather_body, n_pad),
        out_type=jax.ShapeDtypeStruct((n_pad, _H), jnp.float32),
        mesh=mesh,
        scratch_types=[
            pltpu.VMEM((_CH,), jnp.int32),
            pltpu.VMEM((_CH, _H), jnp.float32),
            pltpu.SemaphoreType.DMA,
        ],
    )(table, idx)


def _sc_compose_body(tab_n, n, tab, idx, out, tab_v, idx_v, out_v):
    per_w = n // _NW
    base = _wid() * per_w
    pltpu.sync_copy(tab, tab_v)
    pltpu.sync_copy(idx.at[pl.ds(base, per_w)], idx_v)

    def step(k, carry):
        iv = idx_v[pl.ds(k * 16, 16)]
        out_v[pl.ds(k * 16, 16)] = plsc.load_gather(tab_v, [iv])
        return carry

    lax.fori_loop(0, per_w // 16, step, 0, unroll=False)
    pltpu.sync_copy(out_v, out.at[pl.ds(base, per_w)])


def _sc_compose(tab, idx):
    tab_n = tab.shape[0]
    n = idx.shape[0]
    per_w = n // _NW
    mesh = plsc.VectorSubcoreMesh(core_axis_name="c", subcore_axis_name="s")
    return pl.kernel(
        functools.partial(_sc_compose_body, tab_n, n),
        out_type=jax.ShapeDtypeStruct((n,), jnp.int32),
        mesh=mesh,
        scratch_types=[
            pltpu.VMEM((tab_n,), jnp.int32),
            pltpu.VMEM((per_w,), jnp.int32),
            pltpu.VMEM((per_w,), jnp.int32),
        ],
        compiler_params=pltpu.CompilerParams(needs_layout_passes=False),
    )(tab, idx)


def _mlp_imm_kernel(x_ref, w1_ref, b1_ref, w2_ref, b2_ref, big_ref, small_ref):
    t = jnp.tanh(x_ref[...])
    h = jnp.maximum(t * w1_ref[...] + b1_ref[...], 0.0)
    y = jnp.dot(h, w2_ref[...], preferred_element_type=jnp.float32) + b2_ref[...]
    big_ref[...] = y
    small_ref[...] = y


def _mlp1_kernel(x_ref, w1_ref, b1_ref, w2_ref, b2_ref, alias_ref, out_ref):
    h = jnp.maximum(
        jnp.dot(x_ref[...], w1_ref[...], preferred_element_type=jnp.float32)
        + b1_ref[...], 0.0)
    out_ref[...] = (
        jnp.dot(h, w2_ref[...], preferred_element_type=jnp.float32)
        + b2_ref[...])


def _mlp2_kernel(xa_ref, xb_ref, w1a_ref, w1b_ref, b1_ref, w2_ref, b2_ref,
                 out_ref):
    acc = jnp.dot(xa_ref[...], w1a_ref[...], preferred_element_type=jnp.float32)
    acc += jnp.dot(xb_ref[...], w1b_ref[...], preferred_element_type=jnp.float32)
    h = jnp.maximum(acc + b1_ref[...], 0.0)
    out_ref[...] = (
        jnp.dot(h, w2_ref[...], preferred_element_type=jnp.float32)
        + b2_ref[...])


def _mlp2_alias_kernel(xa_ref, xb_ref, w1a_ref, w1b_ref, b1_ref, w2_ref,
                       b2_ref, alias_ref, out_ref):
    _mlp2_kernel(xa_ref, xb_ref, w1a_ref, w1b_ref, b1_ref, w2_ref, b2_ref,
                 out_ref)


def _full(shape):
    return pl.BlockSpec(shape, lambda i: (0, 0))


def kernel(imm, regs, mem_reg0, mem_reg1, mem_imm0, mem_imm1, mnemic,
           mnemic_idx, operand_idx, emb, W_imm1, b_imm1, W_imm2, b_imm2,
           W_reg1, b_reg1, W_reg2, b_reg2, W_mem1, b_mem1, W_mem2, b_mem2,
           W_ins1, b_ins1, W_ins2, b_ins2):
    f32 = jnp.float32
    i32 = jnp.int32
    B, S = mnemic_idx.shape
    n_ins = B * S

    regs = regs.astype(i32)
    mnemic = mnemic.astype(i32)
    mn_idx_flat = mnemic_idx.astype(i32).reshape(-1)
    op_idx_flat = operand_idx.astype(i32).reshape(-1)

    b1_imm = b_imm1.reshape(1, _H)
    b2_imm = b_imm2.reshape(1, _H)
    b1_reg = b_reg1.reshape(1, _H)
    b2_reg = b_reg2.reshape(1, _H)
    b1_mem = b_mem1.reshape(1, _H)
    b2_mem = b_mem2.reshape(1, _H)
    b1_ins = b_ins1.reshape(1, _H)
    b2_ins = b_ins2.reshape(1, _H)

    mn_rows_idx = _sc_compose(mnemic, mn_idx_flat)

    pad_regs = 102400 - _N_REG
    spread = jnp.arange(pad_regs, dtype=i32) * 37 % jnp.int32(emb.shape[0])
    big_idx = jnp.concatenate([regs, spread, mn_rows_idx])
    emb_rows = _sc_gather(emb, big_idx)

    R = 2000

    n_imm_blocks = _N_IMM // R
    buf0, op_imm = pl.pallas_call(
        _mlp_imm_kernel,
        grid=(n_imm_blocks,),
        in_specs=[
            pl.BlockSpec((R, 1), lambda i: (i, 0)),
            _full((1, _H)), _full((1, _H)), _full((_H, _H)), _full((1, _H)),
        ],
        out_specs=[
            pl.BlockSpec((R, _H), lambda i: (i + _N_REG // R, 0)),
            pl.BlockSpec((R, _H), lambda i: (i, 0)),
        ],
        out_shape=[
            jax.ShapeDtypeStruct((_N_OPS, _H), f32),
            jax.ShapeDtypeStruct((_N_IMM, _H), f32),
        ],
    )(imm, W_imm1, b1_imm, W_imm2, b2_imm)

    buf1 = pl.pallas_call(
        _mlp1_kernel,
        grid=(_N_REG // R,),
        in_specs=[
            pl.BlockSpec((R, _H), lambda i: (i, 0)),
            _full((_H, _H)), _full((1, _H)), _full((_H, _H)), _full((1, _H)),
            pl.BlockSpec(memory_space=pl.ANY),
        ],
        out_specs=pl.BlockSpec((R, _H), lambda i: (i, 0)),
        out_shape=jax.ShapeDtypeStruct((_N_OPS, _H), f32),
        input_output_aliases={5: 0},
    )(emb_rows, W_reg1, b1_reg, W_reg2, b2_reg, buf0)

    pad_mem = 102400 - 2 * _N_MEM
    spread_m = jnp.arange(pad_mem, dtype=i32) * 41 % jnp.int32(184320)
    ridx = jnp.concatenate(
        [jnp.stack([mem_reg0.astype(i32), mem_reg1.astype(i32)], 1).reshape(-1),
         spread_m])
    spread_i = jnp.arange(pad_mem, dtype=i32) * 41 % jnp.int32(_N_IMM)
    iidx = jnp.concatenate(
        [jnp.stack([mem_imm0.astype(i32), mem_imm1.astype(i32)], 1).reshape(-1),
         spread_i])
    rcat = _sc_gather(emb_rows, ridx).reshape(-1, 2 * _H)
    icat = _sc_gather(op_imm, iidx).reshape(-1, 2 * _H)

    operands = pl.pallas_call(
        _mlp2_alias_kernel,
        grid=(_N_MEM // R,),
        in_specs=[
            pl.BlockSpec((R, 2 * _H), lambda i: (i, 0)),
            pl.BlockSpec((R, 2 * _H), lambda i: (i, 0)),
            _full((2 * _H, _H)), _full((2 * _H, _H)), _full((1, _H)),
            _full((_H, _H)), _full((1, _H)),
            pl.BlockSpec(memory_space=pl.ANY),
        ],
        out_specs=pl.BlockSpec(
            (R, _H), lambda i: (i + (_N_REG + _N_IMM) // R, 0)),
        out_shape=jax.ShapeDtypeStruct((_N_OPS, _H), f32),
        input_output_aliases={7: 0},
    )(rcat, icat, W_mem1[: 2 * _H], W_mem1[2 * _H:], b1_mem, W_mem2, b2_mem,
      buf1)

    ops_rows = _sc_gather(operands, op_idx_flat)
    ops2d = ops_rows.reshape(n_ins, 4 * _H)

    RD = 2048
    out = pl.pallas_call(
        _mlp2_kernel,
        grid=(n_ins // RD,),
        in_specs=[
            pl.BlockSpec((RD, _H), lambda i: (i + 102400 // RD, 0)),
            pl.BlockSpec((RD, 4 * _H), lambda i: (i, 0)),
            _full((_H, _H)), _full((4 * _H, _H)), _full((1, _H)),
            _full((_H, _H)), _full((1, _H)),
        ],
        out_specs=pl.BlockSpec((RD, _H), lambda i: (i, 0)),
        out_shape=jax.ShapeDtypeStruct((n_ins, _H), f32),
    )(emb_rows, ops2d, W_ins1[:_H], W_ins1[_H:], b1_ins, W_ins2, b2_ins)

    return out.reshape(B, S, _H)


# speedup vs baseline: 12.1041x; 1.7611x over previous
import functools

import jax
import jax.numpy as jnp
from jax import lax
from jax.experimental import pallas as pl
from jax.experimental.pallas import tpu as pltpu
from jax.experimental.pallas import tpu_sc as plsc

_NC = 2
_NS = 16
_NW = _NC * _NS
_CH = 128
_GRP = 2 * _CH

_H = 128
_N_REG = 100000
_N_IMM = 50000
_N_MEM = 50000
_N_OPS = _N_REG + _N_IMM + _N_MEM


def _wid():
    return lax.axis_index("s") * _NC + lax.axis_index("c")


def _sc_gather_body(n_pad, table, idx, out, idx_v, rows_v, sems):
    b_per_w = n_pad // _NW
    n_groups = b_per_w // _GRP
    base = _wid() * b_per_w

    def fire(g, par):
        pltpu.sync_copy(idx.at[pl.ds(base + g * _GRP, _GRP)], idx_v.at[par])
        for b in range(2):
            pltpu.async_copy(table.at[idx_v.at[par].at[pl.ds(b * _CH, _CH)]],
                             rows_v.at[par].at[pl.ds(b * _CH, _CH)],
                             sems.at[par])

    fire(0, 0)

    def step(j, carry):
        par = lax.rem(j, 2)
        nxt = lax.rem(j + 1, 2)

        @pl.when(j + 1 < n_groups)
        def _():
            fire(j + 1, nxt)

        pltpu.make_async_copy(out.at[pl.ds(0, _GRP)], rows_v.at[par],
                              sems.at[par]).wait()
        pltpu.sync_copy(rows_v.at[par], out.at[pl.ds(base + j * _GRP, _GRP)])
        return carry

    lax.fori_loop(0, n_groups, step, 0, unroll=False)


def _sc_gather(table, idx):
    n_pad = idx.shape[0]
    mesh = plsc.VectorSubcoreMesh(core_axis_name="c", subcore_axis_name="s")
    return pl.kernel(
        functools.partial(_sc_gather_body, n_pad),
        out_type=jax.ShapeDtypeStruct((n_pad, _H), jnp.float32),
        mesh=mesh,
        scratch_types=[
            pltpu.VMEM((2, _GRP), jnp.int32),
            pltpu.VMEM((2, _GRP, _H), jnp.float32),
            pltpu.SemaphoreType.DMA((2,)),
        ],
    )(table, idx)


def _sc_compose_body(tab_n, n, tab, idx, out, tab_v, idx_v, out_v):
    per_w = n // _NW
    base = _wid() * per_w
    pltpu.sync_copy(tab, tab_v)
    pltpu.sync_copy(idx.at[pl.ds(base, per_w)], idx_v)

    def step(k, carry):
        iv = idx_v[pl.ds(k * 16, 16)]
        out_v[pl.ds(k * 16, 16)] = plsc.load_gather(tab_v, [iv])
        return carry

    lax.fori_loop(0, per_w // 16, step, 0, unroll=False)
    pltpu.sync_copy(out_v, out.at[pl.ds(base, per_w)])


def _sc_compose(tab, idx):
    tab_n = tab.shape[0]
    n = idx.shape[0]
    per_w = n // _NW
    mesh = plsc.VectorSubcoreMesh(core_axis_name="c", subcore_axis_name="s")
    return pl.kernel(
        functools.partial(_sc_compose_body, tab_n, n),
        out_type=jax.ShapeDtypeStruct((n,), jnp.int32),
        mesh=mesh,
        scratch_types=[
            pltpu.VMEM((tab_n,), jnp.int32),
            pltpu.VMEM((per_w,), jnp.int32),
            pltpu.VMEM((per_w,), jnp.int32),
        ],
        compiler_params=pltpu.CompilerParams(needs_layout_passes=False),
    )(tab, idx)


def _mlp_imm_kernel(x_ref, w1_ref, b1_ref, w2_ref, b2_ref, big_ref, small_ref):
    t = jnp.tanh(x_ref[...])
    h = jnp.maximum(t * w1_ref[...] + b1_ref[...], 0.0)
    y = jnp.dot(h, w2_ref[...], preferred_element_type=jnp.float32) + b2_ref[...]
    big_ref[...] = y
    small_ref[...] = y


def _mlp1_kernel(x_ref, w1_ref, b1_ref, w2_ref, b2_ref, alias_ref, out_ref):
    h = jnp.maximum(
        jnp.dot(x_ref[...], w1_ref[...], preferred_element_type=jnp.float32)
        + b1_ref[...], 0.0)
    out_ref[...] = (
        jnp.dot(h, w2_ref[...], preferred_element_type=jnp.float32)
        + b2_ref[...])


def _mlp4_alias_kernel(x0, x1, x2, x3, w10, w11, w12, w13, b1_ref, w2_ref,
                       b2_ref, alias_ref, out_ref):
    acc = jnp.dot(x0[...], w10[...], preferred_element_type=jnp.float32)
    acc += jnp.dot(x1[...], w11[...], preferred_element_type=jnp.float32)
    acc += jnp.dot(x2[...], w12[...], preferred_element_type=jnp.float32)
    acc += jnp.dot(x3[...], w13[...], preferred_element_type=jnp.float32)
    h = jnp.maximum(acc + b1_ref[...], 0.0)
    out_ref[...] = (
        jnp.dot(h, w2_ref[...], preferred_element_type=jnp.float32)
        + b2_ref[...])


def _mlp5_kernel(x0, x1, x2, x3, x4, w10, w11, w12, w13, w14, b1_ref, w2_ref,
                 b2_ref, out_ref):
    acc = jnp.dot(x0[...], w10[...], preferred_element_type=jnp.float32)
    acc += jnp.dot(x1[...], w11[...], preferred_element_type=jnp.float32)
    acc += jnp.dot(x2[...], w12[...], preferred_element_type=jnp.float32)
    acc += jnp.dot(x3[...], w13[...], preferred_element_type=jnp.float32)
    acc += jnp.dot(x4[...], w14[...], preferred_element_type=jnp.float32)
    h = jnp.maximum(acc + b1_ref[...], 0.0)
    out_ref[...] = (
        jnp.dot(h, w2_ref[...], preferred_element_type=jnp.float32)
        + b2_ref[...])


def _full(shape):
    return pl.BlockSpec(shape, lambda i: (0, 0))


def _spread(n, mod):
    return jnp.arange(n, dtype=jnp.int32) * 37 % jnp.int32(mod)


def kernel(imm, regs, mem_reg0, mem_reg1, mem_imm0, mem_imm1, mnemic,
           mnemic_idx, operand_idx, emb, W_imm1, b_imm1, W_imm2, b_imm2,
           W_reg1, b_reg1, W_reg2, b_reg2, W_mem1, b_mem1, W_mem2, b_mem2,
           W_ins1, b_ins1, W_ins2, b_ins2):
    f32 = jnp.float32
    i32 = jnp.int32
    B, S = mnemic_idx.shape
    n_ins = B * S

    regs = regs.astype(i32)
    mnemic = mnemic.astype(i32)
    mn_idx_flat = mnemic_idx.astype(i32).reshape(-1)
    op_idx_planar = operand_idx.astype(i32).transpose(2, 0, 1).reshape(-1)

    b1_imm = b_imm1.reshape(1, _H)
    b2_imm = b_imm2.reshape(1, _H)
    b1_reg = b_reg1.reshape(1, _H)
    b2_reg = b_reg2.reshape(1, _H)
    b1_mem = b_mem1.reshape(1, _H)
    b2_mem = b_mem2.reshape(1, _H)
    b1_ins = b_ins1.reshape(1, _H)
    b2_ins = b_ins2.reshape(1, _H)

    mn_rows_idx = _sc_compose(mnemic, mn_idx_flat)

    big_idx = jnp.concatenate([
        regs, _spread(102400 - _N_REG, emb.shape[0]), mn_rows_idx,
        _spread(188416 - 184320, emb.shape[0])])
    emb_rows = _sc_gather(emb, big_idx)
    _MN_OFF = 102400

    R = 2000

    buf0, op_imm = pl.pallas_call(
        _mlp_imm_kernel,
        grid=(_N_IMM // R,),
        in_specs=[
            pl.BlockSpec((R, 1), lambda i: (i, 0)),
            _full((1, _H)), _full((1, _H)), _full((_H, _H)), _full((1, _H)),
        ],
        out_specs=[
            pl.BlockSpec((R, _H), lambda i: (i + _N_REG // R, 0)),
            pl.BlockSpec((R, _H), lambda i: (i, 0)),
        ],
        out_shape=[
            jax.ShapeDtypeStruct((_N_OPS, _H), f32),
            jax.ShapeDtypeStruct((_N_IMM, _H), f32),
        ],
    )(imm, W_imm1, b1_imm, W_imm2, b2_imm)

    buf1 = pl.pallas_call(
        _mlp1_kernel,
        grid=(_N_REG // R,),
        in_specs=[
            pl.BlockSpec((R, _H), lambda i: (i, 0)),
            _full((_H, _H)), _full((1, _H)), _full((_H, _H)), _full((1, _H)),
            pl.BlockSpec(memory_space=pl.ANY),
        ],
        out_specs=pl.BlockSpec((R, _H), lambda i: (i, 0)),
        out_shape=jax.ShapeDtypeStruct((_N_OPS, _H), f32),
        input_output_aliases={5: 0},
    )(emb_rows, W_reg1, b1_reg, W_reg2, b2_reg, buf0)

    _SL = 52000
    ridx = jnp.concatenate([
        mem_reg0.astype(i32), _spread(_SL - _N_MEM, _N_REG),
        mem_reg1.astype(i32), _spread(106496 - _SL - _N_MEM, _N_REG)])
    iidx = jnp.concatenate([
        mem_imm0.astype(i32), _spread(_SL - _N_MEM, _N_IMM),
        mem_imm1.astype(i32), _spread(106496 - _SL - _N_MEM, _N_IMM)])
    rcat = _sc_gather(emb_rows, ridx)
    icat = _sc_gather(op_imm, iidx)

    operands = pl.pallas_call(
        _mlp4_alias_kernel,
        grid=(_N_MEM // R,),
        in_specs=[
            pl.BlockSpec((R, _H), lambda i: (i, 0)),
            pl.BlockSpec((R, _H), lambda i: (i + _SL // R, 0)),
            pl.BlockSpec((R, _H), lambda i: (i, 0)),
            pl.BlockSpec((R, _H), lambda i: (i + _SL // R, 0)),
            _full((_H, _H)), _full((_H, _H)), _full((_H, _H)), _full((_H, _H)),
            _full((1, _H)), _full((_H, _H)), _full((1, _H)),
            pl.BlockSpec(memory_space=pl.ANY),
        ],
        out_specs=pl.BlockSpec(
            (R, _H), lambda i: (i + (_N_REG + _N_IMM) // R, 0)),
        out_shape=jax.ShapeDtypeStruct((_N_OPS, _H), f32),
        input_output_aliases={11: 0},
    )(rcat, rcat, icat, icat, W_mem1[:_H], W_mem1[_H:2 * _H],
      W_mem1[2 * _H:3 * _H], W_mem1[3 * _H:], b1_mem, W_mem2, b2_mem, buf1)

    ops_rows = _sc_gather(operands, op_idx_planar)

    RD = 2048
    n_blk = n_ins // RD
    out = pl.pallas_call(
        _mlp5_kernel,
        grid=(n_blk,),
        in_specs=[
            pl.BlockSpec((RD, _H), lambda i: (i + _MN_OFF // RD, 0)),
            pl.BlockSpec((RD, _H), lambda i: (i, 0)),
            pl.BlockSpec((RD, _H), lambda i: (i + n_ins // RD, 0)),
            pl.BlockSpec((RD, _H), lambda i: (i + 2 * (n_ins // RD), 0)),
            pl.BlockSpec((RD, _H), lambda i: (i + 3 * (n_ins // RD), 0)),
            _full((_H, _H)), _full((_H, _H)), _full((_H, _H)),
            _full((_H, _H)), _full((_H, _H)),
            _full((1, _H)), _full((_H, _H)), _full((1, _H)),
        ],
        out_specs=pl.BlockSpec((RD, _H), lambda i: (i, 0)),
        out_shape=jax.ShapeDtypeStruct((n_ins, _H), f32),
    )(emb_rows, ops_rows, ops_rows, ops_rows, ops_rows,
      W_ins1[:_H], W_ins1[_H:2 * _H], W_ins1[2 * _H:3 * _H],
      W_ins1[3 * _H:4 * _H], W_ins1[4 * _H:], b1_ins, W_ins2, b2_ins)

    return out.reshape(B, S, _H)


# speedup vs baseline: 13.3747x; 1.1050x over previous
import functools

import jax
import jax.numpy as jnp
from jax import lax
from jax.experimental import pallas as pl
from jax.experimental.pallas import tpu as pltpu
from jax.experimental.pallas import tpu_sc as plsc

_NC = 2
_NS = 16
_NW = _NC * _NS
_CH = 128
_GRP = 2 * _CH

_H = 128
_N_REG = 100000
_N_IMM = 50000
_N_MEM = 50000
_N_OPS = _N_REG + _N_IMM + _N_MEM


def _wid():
    return lax.axis_index("s") * _NC + lax.axis_index("c")


def _sc_gather_body(n_pad, table, idx, out, idx_v, rows_v, sems):
    b_per_w = n_pad // _NW
    n_groups = b_per_w // _GRP
    base = _wid() * b_per_w

    def fire(g, par):
        pltpu.sync_copy(idx.at[pl.ds(base + g * _GRP, _GRP)], idx_v.at[par])
        for b in range(2):
            pltpu.async_copy(table.at[idx_v.at[par].at[pl.ds(b * _CH, _CH)]],
                             rows_v.at[par].at[pl.ds(b * _CH, _CH)],
                             sems.at[par])

    fire(0, 0)

    def step(j, carry):
        par = lax.rem(j, 2)
        nxt = lax.rem(j + 1, 2)

        @pl.when(j + 1 < n_groups)
        def _():
            fire(j + 1, nxt)

        pltpu.make_async_copy(out.at[pl.ds(0, _GRP)], rows_v.at[par],
                              sems.at[par]).wait()
        pltpu.sync_copy(rows_v.at[par], out.at[pl.ds(base + j * _GRP, _GRP)])
        return carry

    lax.fori_loop(0, n_groups, step, 0, unroll=False)


def _sc_gather(table, idx):
    n_pad = idx.shape[0]
    mesh = plsc.VectorSubcoreMesh(core_axis_name="c", subcore_axis_name="s")
    return pl.kernel(
        functools.partial(_sc_gather_body, n_pad),
        out_type=jax.ShapeDtypeStruct((n_pad, _H), jnp.float32),
        mesh=mesh,
        scratch_types=[
            pltpu.VMEM((2, _GRP), jnp.int32),
            pltpu.VMEM((2, _GRP, _H), jnp.float32),
            pltpu.SemaphoreType.DMA((2,)),
        ],
    )(table, idx)


def _sc_compose_body(tab_n, n, tab, idx, out, tab_v, idx_v, out_v):
    per_w = n // _NW
    base = _wid() * per_w
    pltpu.sync_copy(tab, tab_v)
    pltpu.sync_copy(idx.at[pl.ds(base, per_w)], idx_v)

    def step(k, carry):
        iv = idx_v[pl.ds(k * 16, 16)]
        out_v[pl.ds(k * 16, 16)] = plsc.load_gather(tab_v, [iv])
        return carry

    lax.fori_loop(0, per_w // 16, step, 0, unroll=False)
    pltpu.sync_copy(out_v, out.at[pl.ds(base, per_w)])


def _sc_compose(tab, idx):
    tab_n = tab.shape[0]
    n = idx.shape[0]
    per_w = n // _NW
    mesh = plsc.VectorSubcoreMesh(core_axis_name="c", subcore_axis_name="s")
    return pl.kernel(
        functools.partial(_sc_compose_body, tab_n, n),
        out_type=jax.ShapeDtypeStruct((n,), jnp.int32),
        mesh=mesh,
        scratch_types=[
            pltpu.VMEM((tab_n,), jnp.int32),
            pltpu.VMEM((per_w,), jnp.int32),
            pltpu.VMEM((per_w,), jnp.int32),
        ],
        compiler_params=pltpu.CompilerParams(needs_layout_passes=False),
    )(tab, idx)


def _mlp_imm_kernel(x_ref, w1_ref, b1_ref, w2_ref, b2_ref, big_ref, small_ref):
    t = jnp.tanh(x_ref[...])
    h = jnp.maximum(t * w1_ref[...] + b1_ref[...], 0.0)
    y = jnp.dot(h, w2_ref[...], preferred_element_type=jnp.float32) + b2_ref[...]
    big_ref[...] = y
    small_ref[...] = y


def _mlp1_kernel(x_ref, w1_ref, b1_ref, w2_ref, b2_ref, alias_ref, out_ref):
    h = jnp.maximum(
        jnp.dot(x_ref[...], w1_ref[...], preferred_element_type=jnp.float32)
        + b1_ref[...], 0.0)
    out_ref[...] = (
        jnp.dot(h, w2_ref[...], preferred_element_type=jnp.float32)
        + b2_ref[...])


def _mlp4_alias_kernel(x0, x1, x2, x3, w10, w11, w12, w13, b1_ref, w2_ref,
                       b2_ref, alias_ref, out_ref):
    acc = jnp.dot(x0[...], w10[...], preferred_element_type=jnp.float32)
    acc += jnp.dot(x1[...], w11[...], preferred_element_type=jnp.float32)
    acc += jnp.dot(x2[...], w12[...], preferred_element_type=jnp.float32)
    acc += jnp.dot(x3[...], w13[...], preferred_element_type=jnp.float32)
    h = jnp.maximum(acc + b1_ref[...], 0.0)
    out_ref[...] = (
        jnp.dot(h, w2_ref[...], preferred_element_type=jnp.float32)
        + b2_ref[...])


def _mlp5_kernel(x0, x1, x2, x3, x4, w10, w11, w12, w13, w14, b1_ref, w2_ref,
                 b2_ref, alias_ref, out_ref):
    acc = jnp.dot(x0[...], w10[...], preferred_element_type=jnp.float32)
    acc += jnp.dot(x1[...], w11[...], preferred_element_type=jnp.float32)
    acc += jnp.dot(x2[...], w12[...], preferred_element_type=jnp.float32)
    acc += jnp.dot(x3[...], w13[...], preferred_element_type=jnp.float32)
    acc += jnp.dot(x4[...], w14[...], preferred_element_type=jnp.float32)
    h = jnp.maximum(acc + b1_ref[...], 0.0)
    y = (jnp.dot(h, w2_ref[...], preferred_element_type=jnp.float32)
         + b2_ref[...])
    out_ref[...] = y.reshape(out_ref.shape)


def _full(shape):
    return pl.BlockSpec(shape, lambda i: (0, 0))


def _spread(n, mod):
    return jnp.arange(n, dtype=jnp.int32) * 37 % jnp.int32(mod)


def kernel(imm, regs, mem_reg0, mem_reg1, mem_imm0, mem_imm1, mnemic,
           mnemic_idx, operand_idx, emb, W_imm1, b_imm1, W_imm2, b_imm2,
           W_reg1, b_reg1, W_reg2, b_reg2, W_mem1, b_mem1, W_mem2, b_mem2,
           W_ins1, b_ins1, W_ins2, b_ins2):
    f32 = jnp.float32
    i32 = jnp.int32
    B, S = mnemic_idx.shape
    n_ins = B * S

    regs = regs.astype(i32)
    mnemic = mnemic.astype(i32)
    mn_idx_flat = mnemic_idx.astype(i32).reshape(-1)
    opi = operand_idx.astype(i32)
    op_idx_halves = [
        opi[h * (B // 2):(h + 1) * (B // 2)].transpose(2, 0, 1).reshape(-1)
        for h in range(2)]

    b1_imm = b_imm1.reshape(1, _H)
    b2_imm = b_imm2.reshape(1, _H)
    b1_reg = b_reg1.reshape(1, _H)
    b2_reg = b_reg2.reshape(1, _H)
    b1_mem = b_mem1.reshape(1, _H)
    b2_mem = b_mem2.reshape(1, _H)
    b1_ins = b_ins1.reshape(1, _H)
    b2_ins = b_ins2.reshape(1, _H)

    mn_rows_idx = _sc_compose(mnemic, mn_idx_flat)

    big_idx = jnp.concatenate([
        regs, _spread(102400 - _N_REG, emb.shape[0]), mn_rows_idx,
        _spread(188416 - 184320, emb.shape[0])])
    emb_rows = _sc_gather(emb, big_idx)
    _MN_OFF = 102400

    R = 2000

    buf0, op_imm = pl.pallas_call(
        _mlp_imm_kernel,
        grid=(_N_IMM // R,),
        in_specs=[
            pl.BlockSpec((R, 1), lambda i: (i, 0)),
            _full((1, _H)), _full((1, _H)), _full((_H, _H)), _full((1, _H)),
        ],
        out_specs=[
            pl.BlockSpec((R, _H), lambda i: (i + _N_REG // R, 0)),
            pl.BlockSpec((R, _H), lambda i: (i, 0)),
        ],
        out_shape=[
            jax.ShapeDtypeStruct((_N_OPS, _H), f32),
            jax.ShapeDtypeStruct((_N_IMM, _H), f32),
        ],
    )(imm, W_imm1, b1_imm, W_imm2, b2_imm)

    buf1 = pl.pallas_call(
        _mlp1_kernel,
        grid=(_N_REG // R,),
        in_specs=[
            pl.BlockSpec((R, _H), lambda i: (i, 0)),
            _full((_H, _H)), _full((1, _H)), _full((_H, _H)), _full((1, _H)),
            pl.BlockSpec(memory_space=pl.ANY),
        ],
        out_specs=pl.BlockSpec((R, _H), lambda i: (i, 0)),
        out_shape=jax.ShapeDtypeStruct((_N_OPS, _H), f32),
        input_output_aliases={5: 0},
    )(emb_rows, W_reg1, b1_reg, W_reg2, b2_reg, buf0)

    _SL = 52000
    ridx = jnp.concatenate([
        mem_reg0.astype(i32), _spread(_SL - _N_MEM, _N_REG),
        mem_reg1.astype(i32), _spread(106496 - _SL - _N_MEM, _N_REG)])
    iidx = jnp.concatenate([
        mem_imm0.astype(i32), _spread(_SL - _N_MEM, _N_IMM),
        mem_imm1.astype(i32), _spread(106496 - _SL - _N_MEM, _N_IMM)])
    rcat = _sc_gather(emb_rows, ridx)
    icat = _sc_gather(op_imm, iidx)

    operands = pl.pallas_call(
        _mlp4_alias_kernel,
        grid=(_N_MEM // R,),
        in_specs=[
            pl.BlockSpec((R, _H), lambda i: (i, 0)),
            pl.BlockSpec((R, _H), lambda i: (i + _SL // R, 0)),
            pl.BlockSpec((R, _H), lambda i: (i, 0)),
            pl.BlockSpec((R, _H), lambda i: (i + _SL // R, 0)),
            _full((_H, _H)), _full((_H, _H)), _full((_H, _H)), _full((_H, _H)),
            _full((1, _H)), _full((_H, _H)), _full((1, _H)),
            pl.BlockSpec(memory_space=pl.ANY),
        ],
        out_specs=pl.BlockSpec(
            (R, _H), lambda i: (i + (_N_REG + _N_IMM) // R, 0)),
        out_shape=jax.ShapeDtypeStruct((_N_OPS, _H), f32),
        input_output_aliases={11: 0},
    )(rcat, rcat, icat, icat, W_mem1[:_H], W_mem1[_H:2 * _H],
      W_mem1[2 * _H:3 * _H], W_mem1[3 * _H:], b1_mem, W_mem2, b2_mem, buf1)

    BB = 128
    RB = BB * S
    n_half = n_ins // 2
    out = None
    for h in range(2):
        ops_rows = _sc_gather(operands, op_idx_halves[h])
        mn_blk_off = (_MN_OFF + h * n_half) // RB
        in_specs = [
            pl.BlockSpec((RB, _H),
                         functools.partial(
                             lambda o, i: (i + o, 0), mn_blk_off)),
            pl.BlockSpec((RB, _H), lambda i: (i, 0)),
            pl.BlockSpec((RB, _H), lambda i: (i + n_half // RB, 0)),
            pl.BlockSpec((RB, _H), lambda i: (i + 2 * (n_half // RB), 0)),
            pl.BlockSpec((RB, _H), lambda i: (i + 3 * (n_half // RB), 0)),
            _full((_H, _H)), _full((_H, _H)), _full((_H, _H)),
            _full((_H, _H)), _full((_H, _H)),
            _full((1, _H)), _full((_H, _H)), _full((1, _H)),
        ]
        args = [emb_rows, ops_rows, ops_rows, ops_rows, ops_rows,
                W_ins1[:_H], W_ins1[_H:2 * _H], W_ins1[2 * _H:3 * _H],
                W_ins1[3 * _H:4 * _H], W_ins1[4 * _H:], b1_ins, W_ins2,
                b2_ins]
        io_alias = {}
        if h == 0:
            in_specs.append(pl.BlockSpec((1, 1, 1), lambda i: (0, 0, 0)))
            args.append(jnp.zeros((1, 1, 1), f32))
        else:
            in_specs.append(pl.BlockSpec(memory_space=pl.ANY))
            args.append(out)
            io_alias = {13: 0}
        out = pl.pallas_call(
            _mlp5_kernel,
            grid=(n_half // RB,),
            in_specs=in_specs,
            out_specs=pl.BlockSpec(
                (BB, S, _H),
                functools.partial(lambda o, i: (i + o, 0, 0),
                                  h * (B // 2) // BB)),
            out_shape=jax.ShapeDtypeStruct((B, S, _H), f32),
            input_output_aliases=io_alias,
        )(*args)

    return out
